# baseline (speedup 1.0000x reference)
import functools

import jax
import jax.numpy as jnp
from jax import lax
from jax.experimental import pallas as pl
from jax.experimental.pallas import tpu as pltpu
from jax.experimental.pallas import tpu_sc as plsc


@functools.lru_cache(maxsize=None)
def _make_gather(n_rows: int, dim: int):
    info = plsc.get_sparse_core_info()
    nc, ns = info.num_cores, info.num_subcores
    nw = nc * ns
    assert n_rows % (8 * nw) == 0
    b_per_w = n_rows // nw

    nbuf = 2
    chunk = 80
    assert b_per_w % (chunk * nbuf) == 0
    n_rounds = b_per_w // (chunk * nbuf)

    mesh = plsc.VectorSubcoreMesh(core_axis_name="c", subcore_axis_name="s")

    @functools.partial(
        pl.kernel,
        mesh=mesh,
        out_type=jax.ShapeDtypeStruct((n_rows, dim), jnp.float32),
        scratch_types=[
            pltpu.VMEM((b_per_w,), jnp.int32),
            pltpu.VMEM((nbuf, chunk, dim), jnp.float32),
            pltpu.SemaphoreType.DMA,
            pltpu.SemaphoreType.DMA,
        ],
    )
    def gather_kernel(table_hbm, idx_hbm, out_hbm, idx_v, rows_v, gsem, ssem):
        wid = lax.axis_index("s") * nc + lax.axis_index("c")
        base = wid * b_per_w
        pltpu.sync_copy(idx_hbm.at[pl.ds(base, b_per_w)], idx_v)

        def gather_start(c, b):
            pltpu.async_copy(
                table_hbm.at[idx_v.at[pl.ds(c * chunk, chunk)]],
                rows_v.at[b], gsem)

        def gather_wait(b):
            pltpu.make_async_copy(
                out_hbm.at[pl.ds(0, chunk)], rows_v.at[b], gsem).wait()

        def scatter_start(c, b):
            pltpu.async_copy(
                rows_v.at[b], out_hbm.at[pl.ds(base + c * chunk, chunk)], ssem)

        def scatter_wait(b):
            pltpu.make_async_copy(
                rows_v.at[b], out_hbm.at[pl.ds(0, chunk)], ssem).wait()

        for b in range(nbuf):
            gather_start(b, b)

        def body(i, carry):
            for b in range(nbuf):
                cur = i * nbuf + b
                gather_wait(b)
                scatter_start(cur, b)
            for b in range(nbuf):
                nxt = (i + 1) * nbuf + b

                @pl.when(nxt < n_rounds * nbuf)
                def _():
                    scatter_wait(b)
                    gather_start(nxt, b)

            return carry

        lax.fori_loop(0, n_rounds, body, 0)
        for b in range(nbuf):
            scatter_wait(b)

    return gather_kernel


def kernel(input, embedding_weight):
    b, l = input.shape
    _, dim = embedding_weight.shape
    idx = input.astype(jnp.int32).T.reshape(-1)
    out2d = _make_gather(idx.shape[0], dim)(embedding_weight, idx)
    return out2d.reshape(l, b, dim).transpose(1, 0, 2)


# speedup vs baseline: 1.0010x; 1.0010x over previous
import functools

import jax
import jax.numpy as jnp
from jax import lax
from jax.experimental import pallas as pl
from jax.experimental.pallas import tpu as pltpu
from jax.experimental.pallas import tpu_sc as plsc


@functools.lru_cache(maxsize=None)
def _make_gather(n_rows: int, dim: int):
    info = plsc.get_sparse_core_info()
    nc, ns = info.num_cores, info.num_subcores
    nw = nc * ns
    assert n_rows % (8 * nw) == 0
    b_per_w = n_rows // nw

    nbuf = 4
    chunk = 40
    assert b_per_w % (chunk * nbuf) == 0
    n_rounds = b_per_w // (chunk * nbuf)

    mesh = plsc.VectorSubcoreMesh(core_axis_name="c", subcore_axis_name="s")

    @functools.partial(
        pl.kernel,
        mesh=mesh,
        out_type=jax.ShapeDtypeStruct((n_rows, dim), jnp.float32),
        scratch_types=[
            pltpu.VMEM((b_per_w,), jnp.int32),
            pltpu.VMEM((nbuf, chunk, dim), jnp.float32),
            pltpu.SemaphoreType.DMA,
            pltpu.SemaphoreType.DMA,
        ],
    )
    def gather_kernel(table_hbm, idx_hbm, out_hbm, idx_v, rows_v, gsem, ssem):
        wid = lax.axis_index("s") * nc + lax.axis_index("c")
        base = wid * b_per_w
        pltpu.sync_copy(idx_hbm.at[pl.ds(base, b_per_w)], idx_v)

        def gather_start(c, b):
            pltpu.async_copy(
                table_hbm.at[idx_v.at[pl.ds(c * chunk, chunk)]],
                rows_v.at[b], gsem)

        def gather_wait(b):
            pltpu.make_async_copy(
                out_hbm.at[pl.ds(0, chunk)], rows_v.at[b], gsem).wait()

        def scatter_start(c, b):
            pltpu.async_copy(
                rows_v.at[b], out_hbm.at[pl.ds(base + c * chunk, chunk)], ssem)

        def scatter_wait(b):
            pltpu.make_async_copy(
                rows_v.at[b], out_hbm.at[pl.ds(0, chunk)], ssem).wait()

        for b in range(nbuf):
            gather_start(b, b)

        def body(i, carry):
            for b in range(nbuf):
                cur = i * nbuf + b
                gather_wait(b)
                scatter_start(cur, b)
            for b in range(nbuf):
                nxt = (i + 1) * nbuf + b

                @pl.when(nxt < n_rounds * nbuf)
                def _():
                    scatter_wait(b)
                    gather_start(nxt, b)

            return carry

        lax.fori_loop(0, n_rounds, body, 0)
        for b in range(nbuf):
            scatter_wait(b)

    return gather_kernel


def kernel(input, embedding_weight):
    b, l = input.shape
    _, dim = embedding_weight.shape
    idx = input.astype(jnp.int32).T.reshape(-1)
    out2d = _make_gather(idx.shape[0], dim)(embedding_weight, idx)
    return out2d.reshape(l, b, dim).transpose(1, 0, 2)
